# baseline (speedup 1.0000x reference)
import functools

import jax
import jax.numpy as jnp
from jax import lax
from jax.experimental import pallas as pl
from jax.experimental.pallas import tpu as pltpu
from jax.experimental.pallas import tpu_sc as plsc

NUM_HEAD = 16
NUM_BUCKETS = 32
MAX_DISTANCE = 128
Q_LEN = 2048
K_LEN = 2048
WD = 4096
SW = 3968
BQ = 2048

_SC_INFO = plsc.get_sparse_core_info()
_NC = _SC_INFO.num_cores
_NW = _NC * _SC_INFO.num_subcores
_CH = WD // _NW

_sc_mesh = plsc.VectorSubcoreMesh(core_axis_name="c", subcore_axis_name="s")


@functools.partial(
    pl.kernel,
    mesh=_sc_mesh,
    out_type=jax.ShapeDtypeStruct((WD, 128), jnp.float32),
    scratch_types=[
        pltpu.VMEM((_CH,), jnp.int32),
        pltpu.VMEM((_CH, 128), jnp.float32),
        pltpu.SemaphoreType.DMA,
    ],
)
def _sc_lookup(d_hbm, w_hbm, out_hbm, idx_v, rows_v, sem):
    wid = lax.axis_index("s") * _NC + lax.axis_index("c")
    base = wid * _CH
    pltpu.sync_copy(d_hbm.at[pl.ds(base, _CH)], idx_v)
    for i in range(_CH // 16):
        d = idx_v[pl.ds(16 * i, 16)]
        a = jnp.abs(d)
        n = jnp.zeros((16,), jnp.int32)
        for thr in (10, 13, 21, 42, 99):
            n = n + jnp.where(a >= thr, 1, 0)
        mid = jnp.where(a < MAX_DISTANCE, 8 + n, 15)
        b = jnp.where(a <= 8, a, mid)
        b = b + jnp.where(d > 0, 16, 0)
        idx_v[pl.ds(16 * i, 16)] = b
    pltpu.async_copy(w_hbm.at[idx_v], rows_v, sem).wait()
    pltpu.sync_copy(rows_v, out_hbm.at[pl.ds(base, _CH)])


def _tc_body(ur_ref, out_ref, s_ref, urt_ref):
    h = pl.program_id(0)
    m = pl.program_id(1)

    @pl.when((h == 0) & (m == 0))
    def _transpose():
        urt_ref[:, 0, :] = jnp.transpose(ur_ref[:, :NUM_HEAD], (1, 0))

    @pl.when(m == 0)
    def _build():
        acc = urt_ref[h]
        for k in range(16):
            rows = [
                jax.lax.slice(acc, (0, 127 - 8 * k - s3), (1, 127 - 8 * k - s3 + SW))
                for s3 in range(8)
            ]
            s_ref[8 * k:8 * k + 8, :] = jnp.concatenate(rows, axis=0)

    for g in range(BQ // 128):
        c0 = pl.multiple_of((Q_LEN - 128) - BQ * m - 128 * g, 128)
        out_ref[0, 0, 128 * g:128 * g + 128, :] = s_ref[:, pl.ds(c0, K_LEN)]


def kernel(q_len, k_len, W):
    delta = jnp.asarray(q_len - k_len, jnp.int32)
    t = jnp.arange(WD, dtype=jnp.int32)
    d_vec = (Q_LEN - 1) - t + delta
    w_pad = jnp.zeros((NUM_BUCKETS, 128), jnp.float32).at[:, :NUM_HEAD].set(
        W.astype(jnp.float32))
    ur_rows = _sc_lookup(d_vec, w_pad)

    out = pl.pallas_call(
        _tc_body,
        grid=(NUM_HEAD, Q_LEN // BQ),
        in_specs=[pl.BlockSpec((WD, 128), lambda h, m: (0, 0))],
        out_specs=pl.BlockSpec((1, 1, BQ, K_LEN), lambda h, m: (0, h, m, 0)),
        out_shape=jax.ShapeDtypeStruct((1, NUM_HEAD, Q_LEN, K_LEN), jnp.float32),
        scratch_shapes=[
            pltpu.VMEM((128, SW), jnp.float32),
            pltpu.VMEM((NUM_HEAD, 1, WD), jnp.float32),
        ],
        compiler_params=pltpu.CompilerParams(
            dimension_semantics=("parallel", "arbitrary")),
    )(ur_rows)
    return out


# speedup vs baseline: 1.4603x; 1.4603x over previous
import functools

import jax
import jax.numpy as jnp
from jax import lax
from jax.experimental import pallas as pl
from jax.experimental.pallas import tpu as pltpu
from jax.experimental.pallas import tpu_sc as plsc

NUM_HEAD = 16
NUM_BUCKETS = 32
MAX_DISTANCE = 128
Q_LEN = 2048
K_LEN = 2048
WD = 4096
SW = 3968
BQ = 2048

_SC_INFO = plsc.get_sparse_core_info()
_NC = _SC_INFO.num_cores
_NW = _NC * _SC_INFO.num_subcores
_CH = WD // _NW

_sc_mesh = plsc.VectorSubcoreMesh(core_axis_name="c", subcore_axis_name="s")


@functools.partial(
    pl.kernel,
    mesh=_sc_mesh,
    out_type=jax.ShapeDtypeStruct((NUM_HEAD, WD), jnp.float32),
    scratch_types=[
        pltpu.VMEM((_CH,), jnp.int32),
        pltpu.VMEM((NUM_HEAD * NUM_BUCKETS,), jnp.float32),
        pltpu.VMEM((NUM_HEAD, _CH), jnp.float32),
    ],
)
def _sc_lookup(d_hbm, wt_hbm, out_hbm, idx_v, wt_v, urt_v):
    wid = lax.axis_index("s") * _NC + lax.axis_index("c")
    base = wid * _CH
    pltpu.sync_copy(wt_hbm, wt_v)
    pltpu.sync_copy(d_hbm.at[pl.ds(base, _CH)], idx_v)
    for i in range(_CH // 16):
        d = idx_v[pl.ds(16 * i, 16)]
        a = jnp.abs(d)
        n = jnp.zeros((16,), jnp.int32)
        for thr in (10, 13, 21, 42, 99):
            n = n + jnp.where(a >= thr, 1, 0)
        mid = jnp.where(a < MAX_DISTANCE, 8 + n, 15)
        b = jnp.where(a <= 8, a, mid)
        b = b + jnp.where(d > 0, 16, 0)
        for h in range(NUM_HEAD):
            acc = jnp.zeros((16,), jnp.float32)
            for bb in range(NUM_BUCKETS):
                w_bc = jnp.broadcast_to(wt_v[pl.ds(h * NUM_BUCKETS + bb, 1)], (16,))
                acc = jnp.where(b == bb, w_bc, acc)
            urt_v[h, pl.ds(16 * i, 16)] = acc
    pltpu.sync_copy(urt_v, out_hbm.at[:, pl.ds(base, _CH)])


def _tc_body(ur_ref, out_ref, s_ref):
    m = pl.program_id(1)

    @pl.when(m == 0)
    def _build():
        acc = ur_ref[0]
        for k in range(16):
            rows = [
                jax.lax.slice(acc, (0, 127 - 8 * k - s3), (1, 127 - 8 * k - s3 + SW))
                for s3 in range(8)
            ]
            s_ref[8 * k:8 * k + 8, :] = jnp.concatenate(rows, axis=0)

    for g in range(BQ // 128):
        c0 = pl.multiple_of((Q_LEN - 128) - BQ * m - 128 * g, 128)
        out_ref[0, 0, 128 * g:128 * g + 128, :] = s_ref[:, pl.ds(c0, K_LEN)]


def kernel(q_len, k_len, W):
    delta = jnp.asarray(q_len - k_len, jnp.int32)
    t = jnp.arange(WD, dtype=jnp.int32)
    d_vec = (Q_LEN - 1) - t + delta
    wt = W.T.astype(jnp.float32).reshape(-1)
    urt = _sc_lookup(d_vec, wt)
    urt3 = urt.reshape(NUM_HEAD, 1, WD)

    out = pl.pallas_call(
        _tc_body,
        grid=(NUM_HEAD, Q_LEN // BQ),
        in_specs=[pl.BlockSpec((1, 1, WD), lambda h, m: (h, 0, 0))],
        out_specs=pl.BlockSpec((1, 1, BQ, K_LEN), lambda h, m: (0, h, m, 0)),
        out_shape=jax.ShapeDtypeStruct((1, NUM_HEAD, Q_LEN, K_LEN), jnp.float32),
        scratch_shapes=[pltpu.VMEM((128, SW), jnp.float32)],
        compiler_params=pltpu.CompilerParams(
            dimension_semantics=("parallel", "arbitrary")),
    )(urt3)
    return out


# speedup vs baseline: 1.5496x; 1.0612x over previous
import functools

import jax
import jax.numpy as jnp
from jax import lax
from jax.experimental import pallas as pl
from jax.experimental.pallas import tpu as pltpu
from jax.experimental.pallas import tpu_sc as plsc

NUM_HEAD = 16
NUM_BUCKETS = 32
MAX_DISTANCE = 128
Q_LEN = 2048
K_LEN = 2048
WD = 4096
SW = 3968
BQ = 2048

_SC_INFO = plsc.get_sparse_core_info()
_NC = _SC_INFO.num_cores
_NW = _NC * _SC_INFO.num_subcores
_CH = WD // _NW

_sc_mesh = plsc.VectorSubcoreMesh(core_axis_name="c", subcore_axis_name="s")


@functools.partial(
    pl.kernel,
    mesh=_sc_mesh,
    out_type=jax.ShapeDtypeStruct((NUM_HEAD, WD), jnp.float32),
    scratch_types=[
        pltpu.VMEM((_CH,), jnp.int32),
        pltpu.VMEM((NUM_HEAD * NUM_BUCKETS,), jnp.float32),
        pltpu.VMEM((NUM_HEAD, _CH), jnp.float32),
    ],
)
def _sc_lookup(d_hbm, wt_hbm, out_hbm, idx_v, wt_v, urt_v):
    wid = lax.axis_index("s") * _NC + lax.axis_index("c")
    base = wid * _CH
    pltpu.sync_copy(wt_hbm, wt_v)
    pltpu.sync_copy(d_hbm.at[pl.ds(base, _CH)], idx_v)
    bv = []
    for i in range(_CH // 16):
        d = idx_v[pl.ds(16 * i, 16)]
        a = jnp.abs(d)
        n = jnp.zeros((16,), jnp.int32)
        for thr in (10, 13, 21, 42, 99):
            n = n + jnp.where(a >= thr, 1, 0)
        mid = jnp.where(a < MAX_DISTANCE, 8 + n, 15)
        b = jnp.where(a <= 8, a, mid)
        bv.append(b + jnp.where(d > 0, 16, 0))
    for h in range(NUM_HEAD):
        w_bcs = [
            jnp.broadcast_to(wt_v[pl.ds(h * NUM_BUCKETS + bb, 1)], (16,))
            for bb in range(NUM_BUCKETS)
        ]
        for i in range(_CH // 16):
            acc = jnp.zeros((16,), jnp.float32)
            for bb in range(NUM_BUCKETS):
                acc = jnp.where(bv[i] == bb, w_bcs[bb], acc)
            urt_v[h, pl.ds(16 * i, 16)] = acc
    pltpu.sync_copy(urt_v, out_hbm.at[:, pl.ds(base, _CH)])


def _tc_body(ur_ref, out_ref, s_ref):
    m = pl.program_id(1)

    @pl.when(m == 0)
    def _build():
        acc = ur_ref[0]
        for k in range(16):
            rows = [
                jax.lax.slice(acc, (0, 127 - 8 * k - s3), (1, 127 - 8 * k - s3 + SW))
                for s3 in range(8)
            ]
            s_ref[8 * k:8 * k + 8, :] = jnp.concatenate(rows, axis=0)

    for g in range(BQ // 128):
        c0 = pl.multiple_of((Q_LEN - 128) - BQ * m - 128 * g, 128)
        out_ref[0, 0, 128 * g:128 * g + 128, :] = s_ref[:, pl.ds(c0, K_LEN)]


def kernel(q_len, k_len, W):
    delta = jnp.asarray(q_len - k_len, jnp.int32)
    t = jnp.arange(WD, dtype=jnp.int32)
    d_vec = (Q_LEN - 1) - t + delta
    wt = W.T.astype(jnp.float32).reshape(-1)
    urt = _sc_lookup(d_vec, wt)
    urt3 = urt.reshape(NUM_HEAD, 1, WD)

    out = pl.pallas_call(
        _tc_body,
        grid=(NUM_HEAD, Q_LEN // BQ),
        in_specs=[pl.BlockSpec((1, 1, WD), lambda h, m: (h, 0, 0))],
        out_specs=pl.BlockSpec((1, 1, BQ, K_LEN), lambda h, m: (0, h, m, 0)),
        out_shape=jax.ShapeDtypeStruct((1, NUM_HEAD, Q_LEN, K_LEN), jnp.float32),
        scratch_shapes=[pltpu.VMEM((128, SW), jnp.float32)],
        compiler_params=pltpu.CompilerParams(
            dimension_semantics=("parallel", "arbitrary")),
    )(urt3)
    return out


# speedup vs baseline: 1.7750x; 1.1455x over previous
import functools

import jax
import jax.numpy as jnp
from jax import lax
from jax.experimental import pallas as pl
from jax.experimental.pallas import tpu as pltpu
from jax.experimental.pallas import tpu_sc as plsc

NUM_HEAD = 16
NUM_BUCKETS = 32
MAX_DISTANCE = 128
Q_LEN = 2048
K_LEN = 2048
WD = 4096
SW = 3968
BQ = 2048

_SC_INFO = plsc.get_sparse_core_info()
_NC = _SC_INFO.num_cores
_NW = _NC * _SC_INFO.num_subcores
_CH = WD // _NW

_sc_mesh = plsc.VectorSubcoreMesh(core_axis_name="c", subcore_axis_name="s")


@functools.partial(
    pl.kernel,
    mesh=_sc_mesh,
    out_type=jax.ShapeDtypeStruct((WD,), jnp.int32),
    scratch_types=[pltpu.VMEM((_CH,), jnp.int32)],
)
def _sc_buckets(d_hbm, out_hbm, idx_v):
    wid = lax.axis_index("s") * _NC + lax.axis_index("c")
    base = wid * _CH
    pltpu.sync_copy(d_hbm.at[pl.ds(base, _CH)], idx_v)
    for i in range(_CH // 16):
        d = idx_v[pl.ds(16 * i, 16)]
        a = jnp.abs(d)
        n = jnp.zeros((16,), jnp.int32)
        for thr in (10, 13, 21, 42, 99):
            n = n + jnp.where(a >= thr, 1, 0)
        mid = jnp.where(a < MAX_DISTANCE, 8 + n, 15)
        b = jnp.where(a <= 8, a, mid)
        idx_v[pl.ds(16 * i, 16)] = b + jnp.where(d > 0, 16, 0)
    pltpu.sync_copy(idx_v, out_hbm.at[pl.ds(base, _CH)])


def _tc_body(b_ref, wt_ref, out_ref, s_ref):
    m = pl.program_id(1)

    @pl.when(m == 0)
    def _build():
        bi = b_ref[:, :]
        wrow = wt_ref[0]
        acc = jnp.zeros((1, WD), jnp.float32)
        for bb in range(NUM_BUCKETS):
            acc = acc + jnp.where(bi == bb, wrow[:, bb:bb + 1], 0.0)
        for k in range(16):
            rows = [
                jax.lax.slice(acc, (0, 127 - 8 * k - s3), (1, 127 - 8 * k - s3 + SW))
                for s3 in range(8)
            ]
            s_ref[8 * k:8 * k + 8, :] = jnp.concatenate(rows, axis=0)

    for g in range(BQ // 128):
        c0 = pl.multiple_of((Q_LEN - 128) - BQ * m - 128 * g, 128)
        out_ref[0, 0, 128 * g:128 * g + 128, :] = s_ref[:, pl.ds(c0, K_LEN)]


def kernel(q_len, k_len, W):
    delta = jnp.asarray(q_len - k_len, jnp.int32)
    t = jnp.arange(WD, dtype=jnp.int32)
    d_vec = (Q_LEN - 1) - t + delta
    buckets = _sc_buckets(d_vec).reshape(1, WD)
    wt = W.T.astype(jnp.float32).reshape(NUM_HEAD, 1, NUM_BUCKETS)

    out = pl.pallas_call(
        _tc_body,
        grid=(NUM_HEAD, Q_LEN // BQ),
        in_specs=[
            pl.BlockSpec((1, WD), lambda h, m: (0, 0)),
            pl.BlockSpec((1, 1, NUM_BUCKETS), lambda h, m: (h, 0, 0)),
        ],
        out_specs=pl.BlockSpec((1, 1, BQ, K_LEN), lambda h, m: (0, h, m, 0)),
        out_shape=jax.ShapeDtypeStruct((1, NUM_HEAD, Q_LEN, K_LEN), jnp.float32),
        scratch_shapes=[pltpu.VMEM((128, SW), jnp.float32)],
        compiler_params=pltpu.CompilerParams(
            dimension_semantics=("parallel", "arbitrary")),
    )(buckets, wt)
    return out


# speedup vs baseline: 1.7829x; 1.0044x over previous
import functools

import jax
import jax.numpy as jnp
from jax import lax
from jax.experimental import pallas as pl
from jax.experimental.pallas import tpu as pltpu
from jax.experimental.pallas import tpu_sc as plsc

NUM_HEAD = 16
NUM_BUCKETS = 32
MAX_DISTANCE = 128
Q_LEN = 2048
K_LEN = 2048
WD = 4096
SW = 3968
BQ = 2048

_SC_INFO = plsc.get_sparse_core_info()
_NC = _SC_INFO.num_cores
_NW = _NC * _SC_INFO.num_subcores
_CH = WD // _NW

_sc_mesh = plsc.VectorSubcoreMesh(core_axis_name="c", subcore_axis_name="s")


@functools.partial(
    pl.kernel,
    mesh=_sc_mesh,
    out_type=jax.ShapeDtypeStruct((WD,), jnp.int32),
    scratch_types=[pltpu.VMEM((_CH,), jnp.int32)],
)
def _sc_buckets(d_hbm, out_hbm, idx_v):
    wid = lax.axis_index("s") * _NC + lax.axis_index("c")
    base = wid * _CH
    pltpu.sync_copy(d_hbm.at[pl.ds(base, _CH)], idx_v)
    for i in range(_CH // 16):
        d = idx_v[pl.ds(16 * i, 16)]
        a = jnp.abs(d)
        n = jnp.zeros((16,), jnp.int32)
        for thr in (10, 13, 21, 42, 99):
            n = n + jnp.where(a >= thr, 1, 0)
        mid = jnp.where(a < MAX_DISTANCE, 8 + n, 15)
        b = jnp.where(a <= 8, a, mid)
        idx_v[pl.ds(16 * i, 16)] = b + jnp.where(d > 0, 16, 0)
    pltpu.sync_copy(idx_v, out_hbm.at[pl.ds(base, _CH)])


def _head_copies(s_ref, out_ref, sem, h, buf):
    cps = []
    for g in range(Q_LEN // 128):
        c0 = pl.multiple_of((Q_LEN - 128) - 128 * g, 128)
        cps.append(pltpu.make_async_copy(
            s_ref.at[buf, :, pl.ds(c0, K_LEN)],
            out_ref.at[0, h, pl.ds(128 * g, 128), :],
            sem.at[buf]))
    return cps


def _tc_body(b_ref, wt_ref, out_ref, s_ref, sem):
    h = pl.program_id(0)
    buf = lax.rem(h, 2)

    @pl.when(h >= 2)
    def _drain():
        for cp in _head_copies(s_ref, out_ref, sem, h - 2, buf):
            cp.wait()

    bi = b_ref[:, :]
    wrow = wt_ref[0]
    acc = jnp.zeros((1, WD), jnp.float32)
    for bb in range(NUM_BUCKETS):
        acc = acc + jnp.where(bi == bb, wrow[:, bb:bb + 1], 0.0)
    for k in range(16):
        rows = [
            jax.lax.slice(acc, (0, 127 - 8 * k - s3), (1, 127 - 8 * k - s3 + SW))
            for s3 in range(8)
        ]
        s_ref[buf, 8 * k:8 * k + 8, :] = jnp.concatenate(rows, axis=0)

    for cp in _head_copies(s_ref, out_ref, sem, h, buf):
        cp.start()

    @pl.when(h == NUM_HEAD - 1)
    def _final_drain():
        for hh in (NUM_HEAD - 2, NUM_HEAD - 1):
            for cp in _head_copies(s_ref, out_ref, sem, hh, lax.rem(hh, 2)):
                cp.wait()


def kernel(q_len, k_len, W):
    delta = jnp.asarray(q_len - k_len, jnp.int32)
    t = jnp.arange(WD, dtype=jnp.int32)
    d_vec = (Q_LEN - 1) - t + delta
    buckets = _sc_buckets(d_vec).reshape(1, WD)
    wt = W.T.astype(jnp.float32).reshape(NUM_HEAD, 1, NUM_BUCKETS)

    out = pl.pallas_call(
        _tc_body,
        grid=(NUM_HEAD,),
        in_specs=[
            pl.BlockSpec((1, WD), lambda h: (0, 0)),
            pl.BlockSpec((1, 1, NUM_BUCKETS), lambda h: (h, 0, 0)),
        ],
        out_specs=pl.BlockSpec(memory_space=pl.ANY),
        out_shape=jax.ShapeDtypeStruct((1, NUM_HEAD, Q_LEN, K_LEN), jnp.float32),
        scratch_shapes=[
            pltpu.VMEM((2, 128, SW), jnp.float32),
            pltpu.SemaphoreType.DMA((2,)),
        ],
    )(buckets, wt)
    return out
